# baseline (speedup 1.0000x reference)
import jax
import jax.numpy as jnp
from jax import lax
from jax.experimental import pallas as pl
from jax.experimental.pallas import tpu as pltpu

_B = 16384
_C = 1000
_BC = 4096
_GRID = _B // _BC


def _onehot_block(ids_ref, o_ref):
    ids = ids_ref[0]
    in_vocab = (ids >= 0) & (ids < _C)
    mapped = jnp.where(in_vocab, ids, _C - 1)
    row = lax.broadcasted_iota(jnp.int32, (_C, _BC), 0)
    o_ref[...] = jnp.where(row == mapped, 1.0, 0.0).astype(jnp.float32)


def kernel(user_ids):
    ids = user_ids.astype(jnp.int32).reshape(_GRID, 1, _BC)
    out_t = pl.pallas_call(
        _onehot_block,
        grid=(_GRID,),
        in_specs=[pl.BlockSpec((1, 1, _BC), lambda j: (j, 0, 0))],
        out_specs=pl.BlockSpec((_C, _BC), lambda j: (0, j)),
        out_shape=jax.ShapeDtypeStruct((_C, _B), jnp.float32),
    )(ids)
    return out_t.T


# speedup vs baseline: 1.1105x; 1.1105x over previous
import jax
import jax.numpy as jnp
from jax import lax
from jax.experimental import pallas as pl
from jax.experimental.pallas import tpu as pltpu

_B = 16384
_C = 1000
_BC = 1024
_GRID = _B // _BC


def _onehot_block(ids_ref, o_ref):
    ids = ids_ref[0]
    in_vocab = (ids >= 0) & (ids < _C)
    mapped = jnp.where(in_vocab, ids, _C - 1)
    row = lax.broadcasted_iota(jnp.int32, (_C, _BC), 0)
    o_ref[...] = jnp.where(row == mapped, 1.0, 0.0).astype(jnp.float32)


def kernel(user_ids):
    ids = user_ids.astype(jnp.int32).reshape(_GRID, 1, _BC)
    out_t = pl.pallas_call(
        _onehot_block,
        grid=(_GRID,),
        in_specs=[pl.BlockSpec((1, 1, _BC), lambda j: (j, 0, 0))],
        out_specs=pl.BlockSpec((_C, _BC), lambda j: (0, j)),
        out_shape=jax.ShapeDtypeStruct((_C, _B), jnp.float32),
    )(ids)
    return out_t.T
